# baseline (speedup 1.0000x reference)
import functools

import jax
import jax.numpy as jnp
from jax import lax
from jax.experimental import pallas as pl
from jax.experimental.pallas import tpu as pltpu
from jax.experimental.pallas import tpu_sc as plsc

N = 10000
E = 320000
D = 128

NC = 2
NS = 16
NW = NC * NS
EPW = E // NW
CB = 125
CK = EPW // CB
GC = 16
NG = CK // GC
NIT = 10
RPT = N // NIT

def _sc_aggregate_body(x_hbm, src_hbm, dst_hbm, zrows_hbm, zflat_hbm, ones_hbm,
                       y0_hbm, y1_hbm, c0_hbm, c1_hbm,
                       srcv, dstv, rows0, rows1, ones, acc, dcnt, sem, semo):
    cid = lax.axis_index("c")
    sid = lax.axis_index("s")
    wid = sid * NC + cid
    rs = pl.ds(sid * RPT, RPT)

    @pl.when(sid < NIT)
    def _():
        pltpu.sync_copy(zrows_hbm.at[rs], acc.at[rs])

    @pl.when(sid == 0)
    def _():
        pltpu.sync_copy(zflat_hbm, dcnt)

    pltpu.sync_copy(ones_hbm, ones)

    plsc.subcore_barrier()

    def group(g, carry):
        gs = pl.ds(g * GC, GC)
        pltpu.sync_copy(src_hbm.at[wid, gs], srcv)
        pltpu.sync_copy(dst_hbm.at[wid, gs], dstv)
        pltpu.async_copy(x_hbm.at[srcv.at[0]], rows0, sem)

        def pair(p, c2):
            j0 = 2 * p
            j1 = j0 + 1
            pltpu.make_async_copy(x_hbm.at[srcv.at[j0]], rows0, sem).wait()
            pltpu.async_copy(x_hbm.at[srcv.at[j1]], rows1, sem)
            pltpu.async_copy(ones, dcnt.at[dstv.at[j0]], semo, add=True)
            pltpu.sync_copy(rows0, acc.at[dstv.at[j0]], add=True)
            pltpu.make_async_copy(x_hbm.at[srcv.at[j1]], rows1, sem).wait()

            @pl.when(p < GC // 2 - 1)
            def _():
                pltpu.async_copy(x_hbm.at[srcv.at[j0 + 2]], rows0, sem)

            pltpu.async_copy(ones, dcnt.at[dstv.at[j1]], semo, add=True)
            pltpu.sync_copy(rows1, acc.at[dstv.at[j1]], add=True)
            return c2

        lax.fori_loop(0, GC // 2, pair, 0)

        def drain(p, c2):
            pltpu.make_async_copy(ones, dcnt.at[dstv.at[0]], semo).wait()
            return c2

        lax.fori_loop(0, GC, drain, 0)
        return carry

    lax.fori_loop(0, NG, group, 0)

    plsc.subcore_barrier()

    @pl.when(cid == 0)
    def _():
        @pl.when(sid < NIT)
        def _():
            pltpu.sync_copy(acc.at[rs], y0_hbm.at[rs])

        @pl.when(sid == 0)
        def _():
            pltpu.sync_copy(dcnt, c0_hbm)

    @pl.when(cid == 1)
    def _():
        @pl.when(sid < NIT)
        def _():
            pltpu.sync_copy(acc.at[rs], y1_hbm.at[rs])

        @pl.when(sid == 0)
        def _():
            pltpu.sync_copy(dcnt, c1_hbm)


@functools.cache
def _sc_aggregate():
    mesh = plsc.VectorSubcoreMesh(core_axis_name="c", subcore_axis_name="s")
    return pl.kernel(
        _sc_aggregate_body,
        out_type=[
            jax.ShapeDtypeStruct((N, D), jnp.float32),
            jax.ShapeDtypeStruct((N, D), jnp.float32),
            jax.ShapeDtypeStruct((N,), jnp.float32),
            jax.ShapeDtypeStruct((N,), jnp.float32),
        ],
        mesh=mesh,
        scratch_types=[
            pltpu.VMEM((GC, CB), jnp.int32),
            pltpu.VMEM((GC, CB), jnp.int32),
            pltpu.VMEM((CB, D), jnp.float32),
            pltpu.VMEM((CB, D), jnp.float32),
            pltpu.VMEM((CB,), jnp.float32),
            pltpu.VMEM_SHARED((N, D), jnp.float32),
            pltpu.VMEM_SHARED((N,), jnp.float32),
            pltpu.SemaphoreType.DMA,
            pltpu.SemaphoreType.DMA,
        ],
    )


BN = 1000


def _mid_body(y0, y1, c0, c1, w1p, b1p, w1t, b1t, w2p, w2t, z, dinv, mv):
    cnt = c0[...] + c1[...]
    di = 1.0 / jnp.maximum(cnt, 1.0)
    m = cnt * di
    y = (y0[...] + y1[...]) * di
    ap = jnp.dot(y, w1p[...], preferred_element_type=jnp.float32) + m * b1p[...]
    at = jnp.dot(y, w1t[...], preferred_element_type=jnp.float32) + m * b1t[...]
    rp = jnp.maximum(ap, 0.0)
    rt = jnp.maximum(at, 0.0)
    z[...] = (jnp.dot(rt, w2t[...], preferred_element_type=jnp.float32)
              - jnp.dot(rp, w2p[...], preferred_element_type=jnp.float32))
    dinv[...] = di
    mv[...] = m


def _loss_body(u0, u1, dinv, mv, b2p, b2t, out):
    i = pl.program_id(0)
    nb = pl.num_programs(0)
    diff = (u0[...] + u1[...]) * dinv[...] + mv[...] * (b2t[...] - b2p[...])
    part = jnp.sum(diff * diff)
    tot = jnp.where(i == 0, part, out[...] + part)
    out[...] = tot * jnp.where(i == nb - 1, 1.0 / (N * D), 1.0)


def _row_spec(bn, w):
    return pl.BlockSpec((bn, w), lambda i: (i, 0))


def _full_spec(a, b):
    return pl.BlockSpec((a, b), lambda i: (0, 0))


_tc_mid = pl.pallas_call(
    _mid_body,
    grid=(N // BN,),
    in_specs=[
        _row_spec(BN, D), _row_spec(BN, D),
        _row_spec(BN, 1), _row_spec(BN, 1),
        _full_spec(D, D), _full_spec(1, D),
        _full_spec(D, D), _full_spec(1, D),
        _full_spec(D, D), _full_spec(D, D),
    ],
    out_specs=[_row_spec(BN, D), _row_spec(BN, 1), _row_spec(BN, 1)],
    out_shape=[
        jax.ShapeDtypeStruct((N, D), jnp.float32),
        jax.ShapeDtypeStruct((N, 1), jnp.float32),
        jax.ShapeDtypeStruct((N, 1), jnp.float32),
    ],
)

_tc_loss = pl.pallas_call(
    _loss_body,
    grid=(N // BN,),
    in_specs=[
        _row_spec(BN, D), _row_spec(BN, D),
        _row_spec(BN, 1), _row_spec(BN, 1),
        _full_spec(1, D), _full_spec(1, D),
    ],
    out_specs=pl.BlockSpec((1, 1), lambda i: (0, 0)),
    out_shape=jax.ShapeDtypeStruct((1, 1), jnp.float32),
)


def kernel(x, edge_index, W1p, b1p, W2p, b2p, W1t, b1t, W2t, b2t):
    src3 = edge_index[0].reshape(NW, CK, CB)
    dst3 = edge_index[1].reshape(NW, CK, CB)
    zrows = jnp.zeros((N, D), jnp.float32)
    zflat = jnp.zeros((N,), jnp.float32)
    ones = jnp.ones((CB,), jnp.float32)

    y0, y1, c0, c1 = _sc_aggregate()(x, src3, dst3, zrows, zflat, ones)
    z, dinv, mv = _tc_mid(y0, y1, c0.reshape(N, 1), c1.reshape(N, 1),
                          W1p, b1p.reshape(1, D), W1t, b1t.reshape(1, D),
                          W2p, W2t)
    u0, u1, _, _ = _sc_aggregate()(z, src3, dst3, zrows, zflat, ones)
    loss = _tc_loss(u0, u1, dinv, mv, b2p.reshape(1, D), b2t.reshape(1, D))
    return loss.reshape(())


# speedup vs baseline: 1.0278x; 1.0278x over previous
import functools

import jax
import jax.numpy as jnp
from jax import lax
from jax.experimental import pallas as pl
from jax.experimental.pallas import tpu as pltpu
from jax.experimental.pallas import tpu_sc as plsc

N = 10000
E = 320000
D = 128

NC = 2
NS = 16
NW = NC * NS
EPW = E // NW
CB = 50
CK = EPW // CB
GC = 8
NG = CK // GC
NB = 4
NIT = 10
RPT = N // NIT

def _sc_aggregate_body(x_hbm, src_hbm, dst_hbm, zrows_hbm, zflat_hbm, ones_hbm,
                       y0_hbm, y1_hbm, c0_hbm, c1_hbm,
                       srcv, dstv, rows0, rows1, rows2, rows3, ones,
                       acc, dcnt, s0, s1, s2, s3, semi, semo):
    rows = (rows0, rows1, rows2, rows3)
    sems = (s0, s1, s2, s3)
    cid = lax.axis_index("c")
    sid = lax.axis_index("s")
    wid = sid * NC + cid
    rs = pl.ds(sid * RPT, RPT)

    @pl.when(sid < NIT)
    def _():
        pltpu.sync_copy(zrows_hbm.at[rs], acc.at[rs])

    @pl.when(sid == 0)
    def _():
        pltpu.sync_copy(zflat_hbm, dcnt)

    pltpu.sync_copy(ones_hbm, ones)

    plsc.subcore_barrier()

    def gather(par, jj, k):
        pltpu.async_copy(x_hbm.at[srcv.at[par, jj]], rows[k], sems[k])

    def wait_gather(par, jj, k):
        pltpu.make_async_copy(x_hbm.at[srcv.at[par, jj]], rows[k], sems[k]).wait()

    def scatter(par, jj, k):
        pltpu.async_copy(rows[k], acc.at[dstv.at[par, jj]], sems[k], add=True)
        pltpu.async_copy(ones, dcnt.at[dstv.at[par, jj]], semo, add=True)

    def wait_scatter(k):
        pltpu.make_async_copy(rows[k], acc.at[dstv.at[0, 0]], sems[k]).wait()

    pltpu.sync_copy(src_hbm.at[wid, pl.ds(0, GC)], srcv.at[0])
    pltpu.sync_copy(dst_hbm.at[wid, pl.ds(0, GC)], dstv.at[0])
    gather(0, 0, 0)
    gather(0, 1, 1)

    def group(g, carry):
        par = lax.rem(g, 2)
        nxt = lax.rem(g + 1, 2)
        for jj in range(GC):
            k = jj % NB
            kp = (jj + 2) % NB
            wait_gather(par, jj, k)
            scatter(par, jj, k)
            if jj == 2:
                @pl.when(g < NG - 1)
                def _():
                    gs = pl.ds((g + 1) * GC, GC)
                    pltpu.async_copy(src_hbm.at[wid, gs], srcv.at[nxt], semi)
                    pltpu.async_copy(dst_hbm.at[wid, gs], dstv.at[nxt], semi)
            if jj < GC - 2:
                if jj < 2:
                    @pl.when(g > 0)
                    def _():
                        wait_scatter(kp)
                else:
                    wait_scatter(kp)
                gather(par, jj + 2, kp)
            else:
                @pl.when(g < NG - 1)
                def _():
                    if jj == GC - 2:
                        pltpu.make_async_copy(
                            src_hbm.at[wid, pl.ds(0, GC)], srcv.at[nxt], semi
                        ).wait()
                        pltpu.make_async_copy(
                            dst_hbm.at[wid, pl.ds(0, GC)], dstv.at[nxt], semi
                        ).wait()
                    wait_scatter(kp)
                    gather(nxt, jj + 2 - GC, kp)

        def drain(p, c2):
            pltpu.make_async_copy(ones, dcnt.at[dstv.at[0, 0]], semo).wait()
            return c2

        lax.fori_loop(0, GC, drain, 0)
        return carry

    lax.fori_loop(0, NG, group, 0)

    for k in range(NB):
        wait_scatter(k)

    plsc.subcore_barrier()

    @pl.when(cid == 0)
    def _():
        @pl.when(sid < NIT)
        def _():
            pltpu.sync_copy(acc.at[rs], y0_hbm.at[rs])

        @pl.when(sid == 0)
        def _():
            pltpu.sync_copy(dcnt, c0_hbm)

    @pl.when(cid == 1)
    def _():
        @pl.when(sid < NIT)
        def _():
            pltpu.sync_copy(acc.at[rs], y1_hbm.at[rs])

        @pl.when(sid == 0)
        def _():
            pltpu.sync_copy(dcnt, c1_hbm)


@functools.cache
def _sc_aggregate():
    mesh = plsc.VectorSubcoreMesh(core_axis_name="c", subcore_axis_name="s")
    return pl.kernel(
        _sc_aggregate_body,
        out_type=[
            jax.ShapeDtypeStruct((N, D), jnp.float32),
            jax.ShapeDtypeStruct((N, D), jnp.float32),
            jax.ShapeDtypeStruct((N,), jnp.float32),
            jax.ShapeDtypeStruct((N,), jnp.float32),
        ],
        mesh=mesh,
        scratch_types=[
            pltpu.VMEM((2, GC, CB), jnp.int32),
            pltpu.VMEM((2, GC, CB), jnp.int32),
            pltpu.VMEM((CB, D), jnp.float32),
            pltpu.VMEM((CB, D), jnp.float32),
            pltpu.VMEM((CB, D), jnp.float32),
            pltpu.VMEM((CB, D), jnp.float32),
            pltpu.VMEM((CB,), jnp.float32),
            pltpu.VMEM_SHARED((N, D), jnp.float32),
            pltpu.VMEM_SHARED((N,), jnp.float32),
            pltpu.SemaphoreType.DMA,
            pltpu.SemaphoreType.DMA,
            pltpu.SemaphoreType.DMA,
            pltpu.SemaphoreType.DMA,
            pltpu.SemaphoreType.DMA,
            pltpu.SemaphoreType.DMA,
        ],
    )


BN = 1000


def _mid_body(y0, y1, c0, c1, w1p, b1p, w1t, b1t, w2p, w2t, z, dinv, mv):
    cnt = c0[...] + c1[...]
    di = 1.0 / jnp.maximum(cnt, 1.0)
    m = cnt * di
    y = (y0[...] + y1[...]) * di
    ap = jnp.dot(y, w1p[...], preferred_element_type=jnp.float32) + m * b1p[...]
    at = jnp.dot(y, w1t[...], preferred_element_type=jnp.float32) + m * b1t[...]
    rp = jnp.maximum(ap, 0.0)
    rt = jnp.maximum(at, 0.0)
    z[...] = (jnp.dot(rt, w2t[...], preferred_element_type=jnp.float32)
              - jnp.dot(rp, w2p[...], preferred_element_type=jnp.float32))
    dinv[...] = di
    mv[...] = m


def _loss_body(u0, u1, dinv, mv, b2p, b2t, out):
    i = pl.program_id(0)
    nb = pl.num_programs(0)
    diff = (u0[...] + u1[...]) * dinv[...] + mv[...] * (b2t[...] - b2p[...])
    part = jnp.sum(diff * diff)
    tot = jnp.where(i == 0, part, out[...] + part)
    out[...] = tot * jnp.where(i == nb - 1, 1.0 / (N * D), 1.0)


def _row_spec(bn, w):
    return pl.BlockSpec((bn, w), lambda i: (i, 0))


def _full_spec(a, b):
    return pl.BlockSpec((a, b), lambda i: (0, 0))


_tc_mid = pl.pallas_call(
    _mid_body,
    grid=(N // BN,),
    in_specs=[
        _row_spec(BN, D), _row_spec(BN, D),
        _row_spec(BN, 1), _row_spec(BN, 1),
        _full_spec(D, D), _full_spec(1, D),
        _full_spec(D, D), _full_spec(1, D),
        _full_spec(D, D), _full_spec(D, D),
    ],
    out_specs=[_row_spec(BN, D), _row_spec(BN, 1), _row_spec(BN, 1)],
    out_shape=[
        jax.ShapeDtypeStruct((N, D), jnp.float32),
        jax.ShapeDtypeStruct((N, 1), jnp.float32),
        jax.ShapeDtypeStruct((N, 1), jnp.float32),
    ],
)

_tc_loss = pl.pallas_call(
    _loss_body,
    grid=(N // BN,),
    in_specs=[
        _row_spec(BN, D), _row_spec(BN, D),
        _row_spec(BN, 1), _row_spec(BN, 1),
        _full_spec(1, D), _full_spec(1, D),
    ],
    out_specs=pl.BlockSpec((1, 1), lambda i: (0, 0)),
    out_shape=jax.ShapeDtypeStruct((1, 1), jnp.float32),
)


def kernel(x, edge_index, W1p, b1p, W2p, b2p, W1t, b1t, W2t, b2t):
    src3 = edge_index[0].reshape(NW, CK, CB)
    dst3 = edge_index[1].reshape(NW, CK, CB)
    zrows = jnp.zeros((N, D), jnp.float32)
    zflat = jnp.zeros((N,), jnp.float32)
    ones = jnp.ones((CB,), jnp.float32)

    y0, y1, c0, c1 = _sc_aggregate()(x, src3, dst3, zrows, zflat, ones)
    z, dinv, mv = _tc_mid(y0, y1, c0.reshape(N, 1), c1.reshape(N, 1),
                          W1p, b1p.reshape(1, D), W1t, b1t.reshape(1, D),
                          W2p, W2t)
    u0, u1, _, _ = _sc_aggregate()(z, src3, dst3, zrows, zflat, ones)
    loss = _tc_loss(u0, u1, dinv, mv, b2p.reshape(1, D), b2t.reshape(1, D))
    return loss.reshape(())
